# baseline (speedup 1.0000x reference)
import jax
import jax.numpy as jnp
from jax.experimental import pallas as pl
from jax.experimental.pallas import tpu as pltpu

B = 16
T = 4096
S = 128
NA = 4
AP = 8
TILE = 2048
NT = T // TILE
NEG = -1e30


def _body(len_ref, s_ref, a_ref, w_ref, b_ref, out_ref, acc1, acc2):
    b = pl.program_id(0)
    k = pl.program_id(1)

    @pl.when(jnp.logical_and(b == 0, k == 0))
    def _init():
        acc1[...] = jnp.zeros_like(acc1)
        acc2[...] = jnp.zeros_like(acc2)

    @pl.when(k * TILE < len_ref[b])
    def _compute():
        x = s_ref[0]
        lt = jax.lax.dot_general(
            w_ref[...], x, (((0,), (1,)), ((), ())),
            preferred_element_type=jnp.float32,
        ) + b_ref[...]
        m = jnp.max(lt, axis=0, keepdims=True)
        ssum = jnp.sum(jnp.exp(lt - m), axis=0, keepdims=True)
        lse = m + jnp.log(ssum)
        acts = a_ref[0]
        rows = jax.lax.broadcasted_iota(jnp.int32, (AP, TILE), 0)
        t_idx = k * TILE + jax.lax.broadcasted_iota(jnp.int32, (1, TILE), 1)
        mask = (t_idx < len_ref[b]).astype(jnp.float32)
        onehot = jnp.where(rows == acts, mask, 0.0)
        acc1[...] += lse * mask
        acc2[...] += lt * onehot

    @pl.when(jnp.logical_and(b == B - 1, k == NT - 1))
    def _final():
        out_ref[0, 0] = jnp.sum(acc1[...]) - jnp.sum(acc2[...])


@jax.jit
def kernel(s_i_batch, actions_batch, lengths, W, bias):
    acts3 = actions_batch.reshape(B, 1, T).astype(jnp.int32)
    w_pad = jnp.zeros((S, AP), jnp.float32).at[:, :NA].set(W)
    b_pad = jnp.full((AP, 1), NEG, jnp.float32).at[:NA, 0].set(bias)
    lens = lengths.astype(jnp.int32)

    def clamp(lens, b, k):
        return jnp.minimum(k, pl.cdiv(lens[b], TILE) - 1)

    grid_spec = pltpu.PrefetchScalarGridSpec(
        num_scalar_prefetch=1,
        grid=(B, NT),
        in_specs=[
            pl.BlockSpec((1, TILE, S), lambda b, k, L: (b, clamp(L, b, k), 0)),
            pl.BlockSpec((1, 1, TILE), lambda b, k, L: (b, 0, clamp(L, b, k))),
            pl.BlockSpec((S, AP), lambda b, k, L: (0, 0)),
            pl.BlockSpec((AP, 1), lambda b, k, L: (0, 0)),
        ],
        out_specs=pl.BlockSpec(
            (1, 1), lambda b, k, L: (0, 0), memory_space=pltpu.SMEM
        ),
        scratch_shapes=[
            pltpu.VMEM((1, TILE), jnp.float32),
            pltpu.VMEM((AP, TILE), jnp.float32),
        ],
    )
    out = pl.pallas_call(
        _body,
        grid_spec=grid_spec,
        out_shape=jax.ShapeDtypeStruct((1, 1), jnp.float32),
    )(lens, s_i_batch, acts3, w_pad, b_pad)
    return out[0, 0]


# speedup vs baseline: 1.0411x; 1.0411x over previous
import jax
import jax.numpy as jnp
from jax.experimental import pallas as pl
from jax.experimental.pallas import tpu as pltpu

B = 16
T = 4096
S = 128
NA = 4
TILE = 2048
NT = T // TILE


def _body(len_ref, s_ref, a_ref, w_ref, bias_ref, out_ref, acc1, acc2):
    b = pl.program_id(0)
    k = pl.program_id(1)

    @pl.when(jnp.logical_and(b == 0, k == 0))
    def _init():
        acc1[...] = jnp.zeros_like(acc1)
        acc2[...] = jnp.zeros_like(acc2)

    @pl.when(k * TILE < len_ref[b])
    def _compute():
        x = s_ref[0]
        lt = jax.lax.dot_general(
            w_ref[...], x, (((0,), (1,)), ((), ())),
            preferred_element_type=jnp.float32,
        )
        rows1 = jax.lax.broadcasted_iota(jnp.int32, (NA, 1), 0)
        bvec = jnp.full((NA, 1), bias_ref[0])
        for a in range(1, NA):
            bvec = jnp.where(rows1 == a, bias_ref[a], bvec)
        lt = lt + bvec
        m = jnp.max(lt, axis=0, keepdims=True)
        ssum = jnp.sum(jnp.exp(lt - m), axis=0, keepdims=True)
        lse = m + jnp.log(ssum)
        acts = a_ref[0]
        rows = jax.lax.broadcasted_iota(jnp.int32, (NA, TILE), 0)
        t_idx = k * TILE + jax.lax.broadcasted_iota(jnp.int32, (1, TILE), 1)
        mask = (t_idx < len_ref[b]).astype(jnp.float32)
        onehot = jnp.where(rows == acts, mask, 0.0)
        acc1[...] += lse * mask
        acc2[...] += lt * onehot

    @pl.when(jnp.logical_and(b == B - 1, k == NT - 1))
    def _final():
        out_ref[0, 0] = jnp.sum(acc1[...]) - jnp.sum(acc2[...])


@jax.jit
def kernel(s_i_batch, actions_batch, lengths, W, bias):
    acts3 = actions_batch.reshape(B, 1, T)

    def clamp(lens, b, k):
        return jnp.minimum(k, pl.cdiv(lens[b], TILE) - 1)

    grid_spec = pltpu.PrefetchScalarGridSpec(
        num_scalar_prefetch=1,
        grid=(B, NT),
        in_specs=[
            pl.BlockSpec((1, TILE, S), lambda b, k, L: (b, clamp(L, b, k), 0)),
            pl.BlockSpec((1, 1, TILE), lambda b, k, L: (b, 0, clamp(L, b, k))),
            pl.BlockSpec((S, NA), lambda b, k, L: (0, 0)),
            pl.BlockSpec(memory_space=pltpu.SMEM),
        ],
        out_specs=pl.BlockSpec(
            (1, 1), lambda b, k, L: (0, 0), memory_space=pltpu.SMEM
        ),
        scratch_shapes=[
            pltpu.VMEM((1, TILE), jnp.float32),
            pltpu.VMEM((NA, TILE), jnp.float32),
        ],
    )
    out = pl.pallas_call(
        _body,
        grid_spec=grid_spec,
        out_shape=jax.ShapeDtypeStruct((1, 1), jnp.float32),
    )(lengths, s_i_batch, acts3, W, bias)
    return out[0, 0]


# speedup vs baseline: 2.7479x; 2.6394x over previous
import jax
import jax.numpy as jnp
from jax.experimental import pallas as pl
from jax.experimental.pallas import tpu as pltpu

B = 16
T = 4096
S = 128
NA = 4
RB = 4096
NK = T * B // RB
SL = RB // 128
TPB = RB // B


def _body(len_ref, s_ref, a_ref, w_ref, bias_ref, out_ref, acc1, acc2):
    k = pl.program_id(0)

    @pl.when(k == 0)
    def _init():
        acc1[...] = jnp.zeros_like(acc1)
        acc2[...] = jnp.zeros_like(acc2)

    x = s_ref[...]
    lt = jax.lax.dot_general(
        w_ref[...], x, (((1,), (1,)), ((), ())),
        preferred_element_type=jnp.float32,
    )
    rows1 = jax.lax.broadcasted_iota(jnp.int32, (NA, 1), 0)
    bvec = jnp.full((NA, 1), bias_ref[0])
    for a in range(1, NA):
        bvec = jnp.where(rows1 == a, bias_ref[a], bvec)
    lt = lt + bvec
    lt3 = lt.reshape(NA, SL, 128)
    m = jnp.max(lt3, axis=0, keepdims=True)
    ssum = jnp.sum(jnp.exp(lt3 - m), axis=0, keepdims=True)
    lse = m + jnp.log(ssum)

    r_idx = (
        jax.lax.broadcasted_iota(jnp.int32, (1, SL, 128), 1) * 128
        + jax.lax.broadcasted_iota(jnp.int32, (1, SL, 128), 2)
    )
    b_col = jax.lax.bitwise_and(r_idx, B - 1)
    t_col = k * TPB + jax.lax.shift_right_logical(r_idx, 4)
    lenv = jnp.full((1, SL, 128), len_ref[0])
    for b in range(1, B):
        lenv = jnp.where(b_col == b, len_ref[b], lenv)
    mask = t_col < lenv

    acts = a_ref[...]
    arows = jax.lax.broadcasted_iota(jnp.int32, (NA, SL, 128), 0)
    sel = jnp.logical_and(arows == acts, mask)
    acc1[...] += jnp.where(mask, lse, 0.0)
    acc2[...] += jnp.where(sel, lt3, 0.0)

    @pl.when(k == NK - 1)
    def _final():
        out_ref[0, 0] = jnp.sum(acc1[...]) - jnp.sum(acc2[...])


@jax.jit
def kernel(s_i_batch, actions_batch, lengths, W, bias):
    s_flat = jnp.transpose(s_i_batch, (1, 0, 2)).reshape((T + 1) * B, S)
    acts_ti = actions_batch.T.reshape(NK, SL, 128)
    w_t = W.T

    grid_spec = pltpu.PrefetchScalarGridSpec(
        num_scalar_prefetch=1,
        grid=(NK,),
        in_specs=[
            pl.BlockSpec((RB, S), lambda k, L: (k, 0)),
            pl.BlockSpec((1, SL, 128), lambda k, L: (k, 0, 0)),
            pl.BlockSpec((NA, S), lambda k, L: (0, 0)),
            pl.BlockSpec(memory_space=pltpu.SMEM),
        ],
        out_specs=pl.BlockSpec(
            (1, 1), lambda k, L: (0, 0), memory_space=pltpu.SMEM
        ),
        scratch_shapes=[
            pltpu.VMEM((1, SL, 128), jnp.float32),
            pltpu.VMEM((NA, SL, 128), jnp.float32),
        ],
    )
    out = pl.pallas_call(
        _body,
        grid_spec=grid_spec,
        out_shape=jax.ShapeDtypeStruct((1, 1), jnp.float32),
    )(lengths, s_flat, acts_ti, w_t, bias)
    return out[0, 0]


# speedup vs baseline: 3.4350x; 1.2500x over previous
import jax
import jax.numpy as jnp
from jax.experimental import pallas as pl
from jax.experimental.pallas import tpu as pltpu

B = 16
T = 4096
S = 128
NA = 4
RB = 8192
NK = T * B // RB
SL = RB // 128
TPB = RB // B


def _body(len_ref, s_ref, a_ref, w_ref, bias_ref, out_ref, acc1, acc2):
    k = pl.program_id(0)

    @pl.when(k == 0)
    def _init():
        acc1[...] = jnp.zeros_like(acc1)
        acc2[...] = jnp.zeros_like(acc2)

    x = s_ref[...]
    lt = jax.lax.dot_general(
        w_ref[...], x, (((1,), (1,)), ((), ())),
        preferred_element_type=jnp.float32,
    )
    rows1 = jax.lax.broadcasted_iota(jnp.int32, (NA, 1), 0)
    bvec = jnp.full((NA, 1), bias_ref[0])
    for a in range(1, NA):
        bvec = jnp.where(rows1 == a, bias_ref[a], bvec)
    lt = lt + bvec
    lt3 = lt.reshape(NA, SL, 128)
    m = jnp.max(lt3, axis=0, keepdims=True)
    ssum = jnp.sum(jnp.exp(lt3 - m), axis=0, keepdims=True)
    lse = m + jnp.log(ssum)

    r_idx = (
        jax.lax.broadcasted_iota(jnp.int32, (1, SL, 128), 1) * 128
        + jax.lax.broadcasted_iota(jnp.int32, (1, SL, 128), 2)
    )
    b_col = jax.lax.bitwise_and(r_idx, B - 1)
    t_col = k * TPB + jax.lax.shift_right_logical(r_idx, 4)
    lenv = jnp.full((1, SL, 128), len_ref[0])
    for b in range(1, B):
        lenv = jnp.where(b_col == b, len_ref[b], lenv)
    mask = t_col < lenv

    acts = a_ref[...]
    arows = jax.lax.broadcasted_iota(jnp.int32, (NA, SL, 128), 0)
    sel = jnp.logical_and(arows == acts, mask)
    acc1[...] += jnp.where(mask, lse, 0.0)
    acc2[...] += jnp.where(sel, lt3, 0.0)

    @pl.when(k == NK - 1)
    def _final():
        out_ref[0, 0] = jnp.sum(acc1[...]) - jnp.sum(acc2[...])


@jax.jit
def kernel(s_i_batch, actions_batch, lengths, W, bias):
    s_flat = jnp.transpose(s_i_batch, (1, 0, 2)).reshape((T + 1) * B, S)
    acts_ti = actions_batch.T.reshape(NK, SL, 128)
    w_t = W.T

    grid_spec = pltpu.PrefetchScalarGridSpec(
        num_scalar_prefetch=1,
        grid=(NK,),
        in_specs=[
            pl.BlockSpec((RB, S), lambda k, L: (k, 0)),
            pl.BlockSpec((1, SL, 128), lambda k, L: (k, 0, 0)),
            pl.BlockSpec((NA, S), lambda k, L: (0, 0)),
            pl.BlockSpec(memory_space=pltpu.SMEM),
        ],
        out_specs=pl.BlockSpec(
            (1, 1), lambda k, L: (0, 0), memory_space=pltpu.SMEM
        ),
        scratch_shapes=[
            pltpu.VMEM((1, SL, 128), jnp.float32),
            pltpu.VMEM((NA, SL, 128), jnp.float32),
        ],
    )
    out = pl.pallas_call(
        _body,
        grid_spec=grid_spec,
        out_shape=jax.ShapeDtypeStruct((1, 1), jnp.float32),
    )(lengths, s_flat, acts_ti, w_t, bias)
    return out[0, 0]


# speedup vs baseline: 3.7037x; 1.0782x over previous
import jax
import jax.numpy as jnp
from jax.experimental import pallas as pl
from jax.experimental.pallas import tpu as pltpu

B = 16
T = 4096
S = 128
NA = 4
RB = 16384
NK = T * B // RB
SL = RB // 128
TPB = RB // B


def _body(len_ref, s_ref, a_ref, w_ref, bias_ref, out_ref, acc1, acc2):
    k = pl.program_id(0)

    @pl.when(k == 0)
    def _init():
        acc1[...] = jnp.zeros_like(acc1)
        acc2[...] = jnp.zeros_like(acc2)

    x = s_ref[...]
    lt = jax.lax.dot_general(
        w_ref[...], x, (((1,), (1,)), ((), ())),
        preferred_element_type=jnp.float32,
    )
    rows1 = jax.lax.broadcasted_iota(jnp.int32, (NA, 1), 0)
    bvec = jnp.full((NA, 1), bias_ref[0])
    for a in range(1, NA):
        bvec = jnp.where(rows1 == a, bias_ref[a], bvec)
    lt = lt + bvec
    lt3 = lt.reshape(NA, SL, 128)
    m = jnp.max(lt3, axis=0, keepdims=True)
    ssum = jnp.sum(jnp.exp(lt3 - m), axis=0, keepdims=True)
    lse = m + jnp.log(ssum)

    r_idx = (
        jax.lax.broadcasted_iota(jnp.int32, (1, SL, 128), 1) * 128
        + jax.lax.broadcasted_iota(jnp.int32, (1, SL, 128), 2)
    )
    b_col = jax.lax.bitwise_and(r_idx, B - 1)
    t_col = k * TPB + jax.lax.shift_right_logical(r_idx, 4)
    lenv = jnp.full((1, SL, 128), len_ref[0])
    for b in range(1, B):
        lenv = jnp.where(b_col == b, len_ref[b], lenv)
    mask = t_col < lenv

    acts = a_ref[...]
    arows = jax.lax.broadcasted_iota(jnp.int32, (NA, SL, 128), 0)
    sel = jnp.logical_and(arows == acts, mask)
    acc1[...] += jnp.where(mask, lse, 0.0)
    acc2[...] += jnp.where(sel, lt3, 0.0)

    @pl.when(k == NK - 1)
    def _final():
        out_ref[0, 0] = jnp.sum(acc1[...]) - jnp.sum(acc2[...])


@jax.jit
def kernel(s_i_batch, actions_batch, lengths, W, bias):
    s_flat = jnp.transpose(s_i_batch, (1, 0, 2)).reshape((T + 1) * B, S)
    acts_ti = actions_batch.T.reshape(NK, SL, 128)
    w_t = W.T

    grid_spec = pltpu.PrefetchScalarGridSpec(
        num_scalar_prefetch=1,
        grid=(NK,),
        in_specs=[
            pl.BlockSpec((RB, S), lambda k, L: (k, 0)),
            pl.BlockSpec((1, SL, 128), lambda k, L: (k, 0, 0)),
            pl.BlockSpec((NA, S), lambda k, L: (0, 0)),
            pl.BlockSpec(memory_space=pltpu.SMEM),
        ],
        out_specs=pl.BlockSpec(
            (1, 1), lambda k, L: (0, 0), memory_space=pltpu.SMEM
        ),
        scratch_shapes=[
            pltpu.VMEM((1, SL, 128), jnp.float32),
            pltpu.VMEM((NA, SL, 128), jnp.float32),
        ],
    )
    out = pl.pallas_call(
        _body,
        grid_spec=grid_spec,
        out_shape=jax.ShapeDtypeStruct((1, 1), jnp.float32),
    )(lengths, s_flat, acts_ti, w_t, bias)
    return out[0, 0]
